# baseline (speedup 1.0000x reference)
import functools

import jax
import jax.numpy as jnp
from jax import lax
from jax.experimental import pallas as pl
from jax.experimental.pallas import tpu as pltpu
from jax.experimental.pallas import tpu_sc as plsc

BN = 256
KC = 256
NSLICES = 4


def _vq_dist_body(x_ref, w2_ref, x2_ref, z2_ref, idx_ref, losssum_ref):
    nt = pl.program_id(0)
    x = x_ref[...]
    w2 = w2_ref[...]
    k = w2.shape[0]
    x2 = x2_ref[...]
    s2 = lax.dot_general(x, w2, (((1,), (1,)), ((), ())),
                         preferred_element_type=jnp.float32)
    n_chunks = k // KC
    mval = None
    mchunk = None
    for c in range(n_chunks):
        z2c = z2_ref[c * KC:(c + 1) * KC][None, :]
        dc = (x2 + z2c) - s2[:, c * KC:(c + 1) * KC]
        if c == 0:
            mval = dc
            mchunk = jnp.zeros(dc.shape, jnp.float32)
        else:
            take = dc < mval
            mval = jnp.minimum(mval, dc)
            mchunk = jnp.where(take, float(c), mchunk)
    dmin = jnp.min(mval, axis=1, keepdims=True)
    lane = lax.broadcasted_iota(jnp.int32, mval.shape, 1).astype(jnp.float32)
    cand = mchunk * float(KC) + lane
    idxf = jnp.min(jnp.where(mval == dmin, cand, float(k)), axis=1)
    idx_ref[...] = idxf.astype(jnp.int32)

    @pl.when(nt == 0)
    def _():
        losssum_ref[0, 0] = 0.0

    losssum_ref[0, 0] += jnp.sum(dmin)


def _vq_distances_slice(x, w2, x2, z2, si, ns):
    n, d = x.shape
    k, _ = w2.shape
    row0 = si * (ns // BN)
    grid = (ns // BN,)
    return pl.pallas_call(
        _vq_dist_body,
        grid=grid,
        in_specs=[
            pl.BlockSpec((BN, d), lambda i: (row0 + i, 0)),
            pl.BlockSpec((k, d), lambda i: (0, 0)),
            pl.BlockSpec((BN, 1), lambda i: (row0 + i, 0)),
            pl.BlockSpec((k,), lambda i: (0,)),
        ],
        out_specs=[
            pl.BlockSpec((BN,), lambda i: (i,)),
            pl.BlockSpec(memory_space=pltpu.SMEM),
        ],
        out_shape=[
            jax.ShapeDtypeStruct((ns,), jnp.int32),
            jax.ShapeDtypeStruct((1, 1), jnp.float32),
        ],
    )(x, w2, x2, z2)


def _make_sc_gather_slice(d, ns, slice_base):
    info = plsc.get_sparse_core_info()
    nw = info.num_cores * info.num_subcores
    b_per_w = ns // nw
    chunk = min(64, b_per_w)
    n_chunks = b_per_w // chunk
    mesh = plsc.VectorSubcoreMesh(core_axis_name="c", subcore_axis_name="s")

    @functools.partial(
        pl.kernel,
        mesh=mesh,
        out_type=(),
        scratch_types=[
            pltpu.VMEM((b_per_w,), jnp.int32),
            pltpu.VMEM((chunk, d), jnp.float32),
            pltpu.VMEM((chunk, d), jnp.float32),
            pltpu.SemaphoreType.DMA,
            pltpu.SemaphoreType.DMA,
            pltpu.SemaphoreType.DMA,
            pltpu.SemaphoreType.DMA,
        ],
    )
    def gather_kernel(table_hbm, idx_hbm, vq_hbm,
                      idx_v, rows0, rows1, gsem0, gsem1, wsem0, wsem1):
        wid = lax.axis_index("s") * info.num_cores + lax.axis_index("c")
        base = wid * b_per_w
        pltpu.sync_copy(idx_hbm.at[pl.ds(base, b_per_w)], idx_v)

        bufs = (rows0, rows1)
        gsems = (gsem0, gsem1)
        wsems = (wsem0, wsem1)

        def gath(c):
            return pltpu.async_copy(
                table_hbm.at[idx_v.at[pl.ds(c * chunk, chunk)]],
                bufs[c % 2], gsems[c % 2])

        writes = [None, None]
        gathers = [None, None]
        gathers[0] = gath(0)
        for c in range(n_chunks):
            nxt = c + 1
            if nxt < n_chunks:
                if writes[nxt % 2] is not None:
                    writes[nxt % 2].wait()
                gathers[nxt % 2] = gath(nxt)
            gathers[c % 2].wait()
            writes[c % 2] = pltpu.async_copy(
                bufs[c % 2],
                vq_hbm.at[pl.ds(slice_base + base + c * chunk, chunk)],
                wsems[c % 2])
        writes[(n_chunks - 1) % 2].wait()
        if n_chunks >= 2:
            writes[(n_chunks - 2) % 2].wait()

    return gather_kernel


def kernel(x, embed_weight):
    n, dim = x.shape
    k, _ = embed_weight.shape
    ns = n // NSLICES
    x2 = jnp.sum(x ** 2, axis=1, keepdims=True)
    z2 = jnp.sum(embed_weight ** 2, axis=1)
    w2 = embed_weight + embed_weight
    vq_ref = jax.empty_ref(jax.ShapeDtypeStruct((n, dim), jnp.float32))
    losssum = jnp.float32(0.0)
    for si in range(NSLICES):
        idx_s, ls = _vq_distances_slice(x, w2, x2, z2, si, ns)
        losssum = losssum + ls[0, 0]
        _make_sc_gather_slice(dim, ns, si * ns)(embed_weight, idx_s, vq_ref)
    loss = losssum * (1.25 / (n * dim))
    return (jax.freeze(vq_ref), loss)


# speedup vs baseline: 1.1141x; 1.1141x over previous
import functools

import jax
import jax.numpy as jnp
from jax import lax
from jax.experimental import pallas as pl
from jax.experimental.pallas import tpu as pltpu
from jax.experimental.pallas import tpu_sc as plsc

BN = 512
KC = 128
NSLICES = 4


def _vq_dist_body(x_ref, w2_ref, x2_ref, z2_ref, idx_ref, losssum_ref):
    nt = pl.program_id(0)
    x = x_ref[...]
    w2 = w2_ref[...]
    k = w2.shape[0]
    x2 = x2_ref[...]
    s2 = lax.dot_general(x, w2, (((1,), (1,)), ((), ())),
                         preferred_element_type=jnp.float32)
    n_chunks = k // KC
    mval = None
    mchunk = None
    for c in range(n_chunks):
        z2c = z2_ref[c * KC:(c + 1) * KC][None, :]
        dc = (x2 + z2c) - s2[:, c * KC:(c + 1) * KC]
        if c == 0:
            mval = dc
            mchunk = jnp.zeros(dc.shape, jnp.float32)
        else:
            take = dc < mval
            mval = jnp.minimum(mval, dc)
            mchunk = jnp.where(take, float(c), mchunk)
    dmin = jnp.min(mval, axis=1, keepdims=True)
    lane = lax.broadcasted_iota(jnp.int32, mval.shape, 1).astype(jnp.float32)
    cand = mchunk * float(KC) + lane
    idxf = jnp.min(jnp.where(mval == dmin, cand, float(k)), axis=1)
    idx_ref[...] = idxf.astype(jnp.int32)

    @pl.when(nt == 0)
    def _():
        losssum_ref[0, 0] = 0.0

    losssum_ref[0, 0] += jnp.sum(dmin)


def _vq_distances_slice(x, w2, x2, z2, si, ns):
    n, d = x.shape
    k, _ = w2.shape
    row0 = si * (ns // BN)
    grid = (ns // BN,)
    return pl.pallas_call(
        _vq_dist_body,
        grid=grid,
        in_specs=[
            pl.BlockSpec((BN, d), lambda i: (row0 + i, 0)),
            pl.BlockSpec((k, d), lambda i: (0, 0)),
            pl.BlockSpec((BN, 1), lambda i: (row0 + i, 0)),
            pl.BlockSpec((k,), lambda i: (0,)),
        ],
        out_specs=[
            pl.BlockSpec((BN,), lambda i: (i,)),
            pl.BlockSpec(memory_space=pltpu.SMEM),
        ],
        out_shape=[
            jax.ShapeDtypeStruct((ns,), jnp.int32),
            jax.ShapeDtypeStruct((1, 1), jnp.float32),
        ],
    )(x, w2, x2, z2)


def _make_sc_gather_slice(d, ns, slice_base):
    info = plsc.get_sparse_core_info()
    nw = info.num_cores * info.num_subcores
    b_per_w = ns // nw
    chunk = min(64, b_per_w)
    n_chunks = b_per_w // chunk
    mesh = plsc.VectorSubcoreMesh(core_axis_name="c", subcore_axis_name="s")

    @functools.partial(
        pl.kernel,
        mesh=mesh,
        out_type=(),
        scratch_types=[
            pltpu.VMEM((b_per_w,), jnp.int32),
            pltpu.VMEM((chunk, d), jnp.float32),
            pltpu.VMEM((chunk, d), jnp.float32),
            pltpu.SemaphoreType.DMA,
            pltpu.SemaphoreType.DMA,
            pltpu.SemaphoreType.DMA,
            pltpu.SemaphoreType.DMA,
        ],
    )
    def gather_kernel(table_hbm, idx_hbm, vq_hbm,
                      idx_v, rows0, rows1, gsem0, gsem1, wsem0, wsem1):
        wid = lax.axis_index("s") * info.num_cores + lax.axis_index("c")
        base = wid * b_per_w
        pltpu.sync_copy(idx_hbm.at[pl.ds(base, b_per_w)], idx_v)

        bufs = (rows0, rows1)
        gsems = (gsem0, gsem1)
        wsems = (wsem0, wsem1)

        def gath(c):
            return pltpu.async_copy(
                table_hbm.at[idx_v.at[pl.ds(c * chunk, chunk)]],
                bufs[c % 2], gsems[c % 2])

        writes = [None, None]
        gathers = [None, None]
        gathers[0] = gath(0)
        for c in range(n_chunks):
            nxt = c + 1
            if nxt < n_chunks:
                if writes[nxt % 2] is not None:
                    writes[nxt % 2].wait()
                gathers[nxt % 2] = gath(nxt)
            gathers[c % 2].wait()
            writes[c % 2] = pltpu.async_copy(
                bufs[c % 2],
                vq_hbm.at[pl.ds(slice_base + base + c * chunk, chunk)],
                wsems[c % 2])
        writes[(n_chunks - 1) % 2].wait()
        if n_chunks >= 2:
            writes[(n_chunks - 2) % 2].wait()

    return gather_kernel


def kernel(x, embed_weight):
    n, dim = x.shape
    k, _ = embed_weight.shape
    ns = n // NSLICES
    x2 = jnp.sum(x ** 2, axis=1, keepdims=True)
    z2 = jnp.sum(embed_weight ** 2, axis=1)
    w2 = embed_weight + embed_weight
    vq_ref = jax.empty_ref(jax.ShapeDtypeStruct((n, dim), jnp.float32))
    losssum = jnp.float32(0.0)
    for si in range(NSLICES):
        idx_s, ls = _vq_distances_slice(x, w2, x2, z2, si, ns)
        losssum = losssum + ls[0, 0]
        _make_sc_gather_slice(dim, ns, si * ns)(embed_weight, idx_s, vq_ref)
    loss = losssum * (1.25 / (n * dim))
    return (jax.freeze(vq_ref), loss)


# speedup vs baseline: 1.1481x; 1.0305x over previous
import functools

import jax
import jax.numpy as jnp
from jax import lax
from jax.experimental import pallas as pl
from jax.experimental.pallas import tpu as pltpu
from jax.experimental.pallas import tpu_sc as plsc

BN = 512
KC = 128
NSLICES = 2


def _vq_dist_body(x_ref, w_ref, x2_ref, z2_ref, idx_ref, losssum_ref,
                  w2_scr):
    nt = pl.program_id(0)

    @pl.when(nt == 0)
    def _():
        w = w_ref[...]
        w2_scr[...] = w + w

    x = x_ref[...]
    w2 = w2_scr[...]
    k = w2.shape[0]
    x2 = x2_ref[...]
    s2 = lax.dot_general(x, w2, (((1,), (1,)), ((), ())),
                         preferred_element_type=jnp.float32)
    n_chunks = k // KC
    mval = None
    mchunk = None
    for c in range(n_chunks):
        z2c = z2_ref[c * KC:(c + 1) * KC][None, :]
        dc = (x2 + z2c) - s2[:, c * KC:(c + 1) * KC]
        if c == 0:
            mval = dc
            mchunk = jnp.zeros(dc.shape, jnp.float32)
        else:
            take = dc < mval
            mval = jnp.minimum(mval, dc)
            mchunk = jnp.where(take, float(c), mchunk)
    dmin = jnp.min(mval, axis=1, keepdims=True)
    lane = lax.broadcasted_iota(jnp.int32, mval.shape, 1).astype(jnp.float32)
    cand = mchunk * float(KC) + lane
    idxf = jnp.min(jnp.where(mval == dmin, cand, float(k)), axis=1)
    idx_ref[...] = idxf.astype(jnp.int32)

    @pl.when(nt == 0)
    def _():
        losssum_ref[0, 0] = 0.0

    losssum_ref[0, 0] += jnp.sum(dmin)


def _vq_distances_slice(x, w, x2, z2, si, ns):
    n, d = x.shape
    k, _ = w.shape
    row0 = si * (ns // BN)
    grid = (ns // BN,)
    return pl.pallas_call(
        _vq_dist_body,
        grid=grid,
        in_specs=[
            pl.BlockSpec((BN, d), lambda i: (row0 + i, 0)),
            pl.BlockSpec((k, d), lambda i: (0, 0)),
            pl.BlockSpec((BN, 1), lambda i: (row0 + i, 0)),
            pl.BlockSpec((k,), lambda i: (0,)),
        ],
        out_specs=[
            pl.BlockSpec((BN,), lambda i: (i,)),
            pl.BlockSpec(memory_space=pltpu.SMEM),
        ],
        out_shape=[
            jax.ShapeDtypeStruct((ns,), jnp.int32),
            jax.ShapeDtypeStruct((1, 1), jnp.float32),
        ],
        scratch_shapes=[pltpu.VMEM((k, d), jnp.float32)],
    )(x, w, x2, z2)


def _make_sc_gather_slice(d, ns, slice_base):
    info = plsc.get_sparse_core_info()
    nw = info.num_cores * info.num_subcores
    b_per_w = ns // nw
    chunk = min(64, b_per_w)
    n_chunks = b_per_w // chunk
    mesh = plsc.VectorSubcoreMesh(core_axis_name="c", subcore_axis_name="s")

    @functools.partial(
        pl.kernel,
        mesh=mesh,
        out_type=(),
        scratch_types=[
            pltpu.VMEM((b_per_w,), jnp.int32),
            pltpu.VMEM((chunk, d), jnp.float32),
            pltpu.VMEM((chunk, d), jnp.float32),
            pltpu.SemaphoreType.DMA,
            pltpu.SemaphoreType.DMA,
            pltpu.SemaphoreType.DMA,
            pltpu.SemaphoreType.DMA,
        ],
    )
    def gather_kernel(table_hbm, idx_hbm, vq_hbm,
                      idx_v, rows0, rows1, gsem0, gsem1, wsem0, wsem1):
        wid = lax.axis_index("s") * info.num_cores + lax.axis_index("c")
        base = wid * b_per_w
        pltpu.sync_copy(idx_hbm.at[pl.ds(base, b_per_w)], idx_v)

        bufs = (rows0, rows1)
        gsems = (gsem0, gsem1)
        wsems = (wsem0, wsem1)

        def gath(c):
            return pltpu.async_copy(
                table_hbm.at[idx_v.at[pl.ds(c * chunk, chunk)]],
                bufs[c % 2], gsems[c % 2])

        writes = [None, None]
        gathers = [None, None]
        gathers[0] = gath(0)
        for c in range(n_chunks):
            nxt = c + 1
            if nxt < n_chunks:
                if writes[nxt % 2] is not None:
                    writes[nxt % 2].wait()
                gathers[nxt % 2] = gath(nxt)
            gathers[c % 2].wait()
            writes[c % 2] = pltpu.async_copy(
                bufs[c % 2],
                vq_hbm.at[pl.ds(slice_base + base + c * chunk, chunk)],
                wsems[c % 2])
        writes[(n_chunks - 1) % 2].wait()
        if n_chunks >= 2:
            writes[(n_chunks - 2) % 2].wait()

    return gather_kernel


def kernel(x, embed_weight):
    n, dim = x.shape
    k, _ = embed_weight.shape
    ns = n // NSLICES
    x2 = jnp.sum(x ** 2, axis=1, keepdims=True)
    z2 = jnp.sum(embed_weight ** 2, axis=1)
    vq_ref = jax.empty_ref(jax.ShapeDtypeStruct((n, dim), jnp.float32))
    losssum = jnp.float32(0.0)
    for si in range(NSLICES):
        idx_s, ls = _vq_distances_slice(x, embed_weight, x2, z2, si, ns)
        losssum = losssum + ls[0, 0]
        _make_sc_gather_slice(dim, ns, si * ns)(embed_weight, idx_s, vq_ref)
    loss = losssum * (1.25 / (n * dim))
    return (jax.freeze(vq_ref), loss)


# speedup vs baseline: 1.1735x; 1.0222x over previous
import functools

import jax
import jax.numpy as jnp
from jax import lax
from jax.experimental import pallas as pl
from jax.experimental.pallas import tpu as pltpu
from jax.experimental.pallas import tpu_sc as plsc

BN = 512
KC = 128
SLICE_ROWS = (7168, 7168, 2048)


def _vq_dist_body(x_ref, w_ref, x2_ref, z2_ref, idx_ref, losssum_ref,
                  w2_scr):
    nt = pl.program_id(0)

    @pl.when(nt == 0)
    def _():
        w = w_ref[...]
        w2_scr[...] = w + w

    x = x_ref[...]
    w2 = w2_scr[...]
    k = w2.shape[0]
    x2 = x2_ref[...]
    s2 = lax.dot_general(x, w2, (((1,), (1,)), ((), ())),
                         preferred_element_type=jnp.float32)
    n_chunks = k // KC
    mval = None
    mchunk = None
    for c in range(n_chunks):
        z2c = z2_ref[c * KC:(c + 1) * KC][None, :]
        dc = (x2 + z2c) - s2[:, c * KC:(c + 1) * KC]
        if c == 0:
            mval = dc
            mchunk = jnp.zeros(dc.shape, jnp.float32)
        else:
            take = dc < mval
            mval = jnp.minimum(mval, dc)
            mchunk = jnp.where(take, float(c), mchunk)
    dmin = jnp.min(mval, axis=1, keepdims=True)
    lane = lax.broadcasted_iota(jnp.int32, mval.shape, 1).astype(jnp.float32)
    cand = mchunk * float(KC) + lane
    idxf = jnp.min(jnp.where(mval == dmin, cand, float(k)), axis=1)
    idx_ref[...] = idxf.astype(jnp.int32)

    @pl.when(nt == 0)
    def _():
        losssum_ref[0, 0] = 0.0

    losssum_ref[0, 0] += jnp.sum(dmin)


def _vq_distances_slice(x, w, x2, z2, row_base, ns):
    n, d = x.shape
    k, _ = w.shape
    row0 = row_base // BN
    grid = (ns // BN,)
    return pl.pallas_call(
        _vq_dist_body,
        grid=grid,
        in_specs=[
            pl.BlockSpec((BN, d), lambda i: (row0 + i, 0)),
            pl.BlockSpec((k, d), lambda i: (0, 0)),
            pl.BlockSpec((BN, 1), lambda i: (row0 + i, 0)),
            pl.BlockSpec((k,), lambda i: (0,)),
        ],
        out_specs=[
            pl.BlockSpec((BN,), lambda i: (i,)),
            pl.BlockSpec(memory_space=pltpu.SMEM),
        ],
        out_shape=[
            jax.ShapeDtypeStruct((ns,), jnp.int32),
            jax.ShapeDtypeStruct((1, 1), jnp.float32),
        ],
        scratch_shapes=[pltpu.VMEM((k, d), jnp.float32)],
    )(x, w, x2, z2)


def _make_sc_gather_slice(d, ns, slice_base):
    info = plsc.get_sparse_core_info()
    nw = info.num_cores * info.num_subcores
    b_per_w = ns // nw
    chunk = next(c for c in range(min(64, b_per_w), 0, -8)
                 if b_per_w % c == 0)
    n_chunks = b_per_w // chunk
    mesh = plsc.VectorSubcoreMesh(core_axis_name="c", subcore_axis_name="s")

    @functools.partial(
        pl.kernel,
        mesh=mesh,
        out_type=(),
        scratch_types=[
            pltpu.VMEM((b_per_w,), jnp.int32),
            pltpu.VMEM((chunk, d), jnp.float32),
            pltpu.VMEM((chunk, d), jnp.float32),
            pltpu.SemaphoreType.DMA,
            pltpu.SemaphoreType.DMA,
            pltpu.SemaphoreType.DMA,
            pltpu.SemaphoreType.DMA,
        ],
    )
    def gather_kernel(table_hbm, idx_hbm, vq_hbm,
                      idx_v, rows0, rows1, gsem0, gsem1, wsem0, wsem1):
        wid = lax.axis_index("s") * info.num_cores + lax.axis_index("c")
        base = wid * b_per_w
        pltpu.sync_copy(idx_hbm.at[pl.ds(base, b_per_w)], idx_v)

        bufs = (rows0, rows1)
        gsems = (gsem0, gsem1)
        wsems = (wsem0, wsem1)

        def gath(c):
            return pltpu.async_copy(
                table_hbm.at[idx_v.at[pl.ds(c * chunk, chunk)]],
                bufs[c % 2], gsems[c % 2])

        writes = [None, None]
        gathers = [None, None]
        gathers[0] = gath(0)
        for c in range(n_chunks):
            nxt = c + 1
            if nxt < n_chunks:
                if writes[nxt % 2] is not None:
                    writes[nxt % 2].wait()
                gathers[nxt % 2] = gath(nxt)
            gathers[c % 2].wait()
            writes[c % 2] = pltpu.async_copy(
                bufs[c % 2],
                vq_hbm.at[pl.ds(slice_base + base + c * chunk, chunk)],
                wsems[c % 2])
        writes[(n_chunks - 1) % 2].wait()
        if n_chunks >= 2:
            writes[(n_chunks - 2) % 2].wait()

    return gather_kernel


def kernel(x, embed_weight):
    n, dim = x.shape
    k, _ = embed_weight.shape
    x2 = jnp.sum(x ** 2, axis=1, keepdims=True)
    z2 = jnp.sum(embed_weight ** 2, axis=1)
    vq_ref = jax.empty_ref(jax.ShapeDtypeStruct((n, dim), jnp.float32))
    losssum = jnp.float32(0.0)
    row_base = 0
    for ns in SLICE_ROWS:
        idx_s, ls = _vq_distances_slice(x, embed_weight, x2, z2, row_base, ns)
        losssum = losssum + ls[0, 0]
        _make_sc_gather_slice(dim, ns, row_base)(embed_weight, idx_s, vq_ref)
        row_base += ns
    loss = losssum * (1.25 / (n * dim))
    return (jax.freeze(vq_ref), loss)


# speedup vs baseline: 1.1962x; 1.0194x over previous
import functools

import jax
import jax.numpy as jnp
from jax import lax
from jax.experimental import pallas as pl
from jax.experimental.pallas import tpu as pltpu
from jax.experimental.pallas import tpu_sc as plsc

BN = 512
KC = 128
SLICE_ROWS = (12288, 4096)


def _vq_dist_body(x_ref, w_ref, x2_ref, z2_ref, idx_ref, losssum_ref,
                  w2_scr):
    nt = pl.program_id(0)

    @pl.when(nt == 0)
    def _():
        w = w_ref[...]
        w2_scr[...] = w + w

    x = x_ref[...]
    w2 = w2_scr[...]
    k = w2.shape[0]
    x2 = x2_ref[...]
    s2 = lax.dot_general(x, w2, (((1,), (1,)), ((), ())),
                         preferred_element_type=jnp.float32)
    n_chunks = k // KC
    mval = None
    mchunk = None
    for c in range(n_chunks):
        z2c = z2_ref[c * KC:(c + 1) * KC][None, :]
        dc = (x2 + z2c) - s2[:, c * KC:(c + 1) * KC]
        if c == 0:
            mval = dc
            mchunk = jnp.zeros(dc.shape, jnp.float32)
        else:
            take = dc < mval
            mval = jnp.minimum(mval, dc)
            mchunk = jnp.where(take, float(c), mchunk)
    dmin = jnp.min(mval, axis=1, keepdims=True)
    lane = lax.broadcasted_iota(jnp.int32, mval.shape, 1).astype(jnp.float32)
    cand = mchunk * float(KC) + lane
    idxf = jnp.min(jnp.where(mval == dmin, cand, float(k)), axis=1)
    idx_ref[...] = idxf.astype(jnp.int32)

    @pl.when(nt == 0)
    def _():
        losssum_ref[0, 0] = 0.0

    losssum_ref[0, 0] += jnp.sum(dmin)


def _vq_distances_slice(x, w, x2, z2, row_base, ns):
    n, d = x.shape
    k, _ = w.shape
    row0 = row_base // BN
    grid = (ns // BN,)
    return pl.pallas_call(
        _vq_dist_body,
        grid=grid,
        in_specs=[
            pl.BlockSpec((BN, d), lambda i: (row0 + i, 0)),
            pl.BlockSpec((k, d), lambda i: (0, 0)),
            pl.BlockSpec((BN, 1), lambda i: (row0 + i, 0)),
            pl.BlockSpec((k,), lambda i: (0,)),
        ],
        out_specs=[
            pl.BlockSpec((BN,), lambda i: (i,)),
            pl.BlockSpec(memory_space=pltpu.SMEM),
        ],
        out_shape=[
            jax.ShapeDtypeStruct((ns,), jnp.int32),
            jax.ShapeDtypeStruct((1, 1), jnp.float32),
        ],
        scratch_shapes=[pltpu.VMEM((k, d), jnp.float32)],
    )(x, w, x2, z2)


def _make_sc_gather_slice(d, ns, slice_base):
    info = plsc.get_sparse_core_info()
    nw = info.num_cores * info.num_subcores
    b_per_w = ns // nw
    chunk = next(c for c in range(min(64, b_per_w), 0, -8)
                 if b_per_w % c == 0)
    n_chunks = b_per_w // chunk
    mesh = plsc.VectorSubcoreMesh(core_axis_name="c", subcore_axis_name="s")

    @functools.partial(
        pl.kernel,
        mesh=mesh,
        out_type=(),
        scratch_types=[
            pltpu.VMEM((b_per_w,), jnp.int32),
            pltpu.VMEM((chunk, d), jnp.float32),
            pltpu.VMEM((chunk, d), jnp.float32),
            pltpu.SemaphoreType.DMA,
            pltpu.SemaphoreType.DMA,
            pltpu.SemaphoreType.DMA,
            pltpu.SemaphoreType.DMA,
        ],
    )
    def gather_kernel(table_hbm, idx_hbm, vq_hbm,
                      idx_v, rows0, rows1, gsem0, gsem1, wsem0, wsem1):
        wid = lax.axis_index("s") * info.num_cores + lax.axis_index("c")
        base = wid * b_per_w
        pltpu.sync_copy(idx_hbm.at[pl.ds(base, b_per_w)], idx_v)

        bufs = (rows0, rows1)
        gsems = (gsem0, gsem1)
        wsems = (wsem0, wsem1)

        def gath(c):
            return pltpu.async_copy(
                table_hbm.at[idx_v.at[pl.ds(c * chunk, chunk)]],
                bufs[c % 2], gsems[c % 2])

        writes = [None, None]
        gathers = [None, None]
        gathers[0] = gath(0)
        for c in range(n_chunks):
            nxt = c + 1
            if nxt < n_chunks:
                if writes[nxt % 2] is not None:
                    writes[nxt % 2].wait()
                gathers[nxt % 2] = gath(nxt)
            gathers[c % 2].wait()
            writes[c % 2] = pltpu.async_copy(
                bufs[c % 2],
                vq_hbm.at[pl.ds(slice_base + base + c * chunk, chunk)],
                wsems[c % 2])
        writes[(n_chunks - 1) % 2].wait()
        if n_chunks >= 2:
            writes[(n_chunks - 2) % 2].wait()

    return gather_kernel


def kernel(x, embed_weight):
    n, dim = x.shape
    k, _ = embed_weight.shape
    x2 = jnp.sum(x ** 2, axis=1, keepdims=True)
    z2 = jnp.sum(embed_weight ** 2, axis=1)
    vq_ref = jax.empty_ref(jax.ShapeDtypeStruct((n, dim), jnp.float32))
    losssum = jnp.float32(0.0)
    row_base = 0
    for ns in SLICE_ROWS:
        idx_s, ls = _vq_distances_slice(x, embed_weight, x2, z2, row_base, ns)
        losssum = losssum + ls[0, 0]
        _make_sc_gather_slice(dim, ns, row_base)(embed_weight, idx_s, vq_ref)
        row_base += ns
    loss = losssum * (1.25 / (n * dim))
    return (jax.freeze(vq_ref), loss)


# speedup vs baseline: 1.2274x; 1.0260x over previous
import functools

import jax
import jax.numpy as jnp
from jax import lax
from jax.experimental import pallas as pl
from jax.experimental.pallas import tpu as pltpu
from jax.experimental.pallas import tpu_sc as plsc

BN = 512
KC = 128
SLICE_ROWS = (12288, 4096)


def _vq_dist_body(x_ref, w_ref, x2_ref, z2_ref, idx_ref, losssum_ref,
                  w2_scr):
    nt = pl.program_id(0)

    @pl.when(nt == 0)
    def _():
        w = w_ref[...]
        w2_scr[...] = w + w

    x = x_ref[...]
    w2 = w2_scr[...]
    k = w2.shape[0]
    x2 = x2_ref[0, 0, :][:, None]
    s2 = lax.dot_general(x, w2, (((1,), (1,)), ((), ())),
                         preferred_element_type=jnp.float32)
    n_chunks = k // KC
    mval = None
    mchunk = None
    for c in range(n_chunks):
        z2c = z2_ref[c * KC:(c + 1) * KC][None, :]
        dc = (x2 + z2c) - s2[:, c * KC:(c + 1) * KC]
        if c == 0:
            mval = dc
            mchunk = jnp.zeros(dc.shape, jnp.float32)
        else:
            take = dc < mval
            mval = jnp.minimum(mval, dc)
            mchunk = jnp.where(take, float(c), mchunk)
    dmin = jnp.min(mval, axis=1, keepdims=True)
    lane = lax.broadcasted_iota(jnp.int32, mval.shape, 1).astype(jnp.float32)
    cand = mchunk * float(KC) + lane
    idxf = jnp.min(jnp.where(mval == dmin, cand, float(k)), axis=1)
    idx_ref[...] = idxf.astype(jnp.int32)

    @pl.when(nt == 0)
    def _():
        losssum_ref[0, 0] = 0.0

    losssum_ref[0, 0] += jnp.sum(dmin)


def _vq_distances_slice(x, w, x2, z2, row_base, ns):
    n, d = x.shape
    k, _ = w.shape
    row0 = row_base // BN
    grid = (ns // BN,)
    return pl.pallas_call(
        _vq_dist_body,
        grid=grid,
        in_specs=[
            pl.BlockSpec((BN, d), lambda i: (row0 + i, 0)),
            pl.BlockSpec((k, d), lambda i: (0, 0)),
            pl.BlockSpec((1, 1, BN), lambda i: (row0 + i, 0, 0)),
            pl.BlockSpec((k,), lambda i: (0,)),
        ],
        out_specs=[
            pl.BlockSpec((BN,), lambda i: (i,)),
            pl.BlockSpec(memory_space=pltpu.SMEM),
        ],
        out_shape=[
            jax.ShapeDtypeStruct((ns,), jnp.int32),
            jax.ShapeDtypeStruct((1, 1), jnp.float32),
        ],
        scratch_shapes=[pltpu.VMEM((k, d), jnp.float32)],
    )(x, w, x2, z2)


def _make_sc_gather_slice(d, ns, slice_base):
    info = plsc.get_sparse_core_info()
    nw = info.num_cores * info.num_subcores
    b_per_w = ns // nw
    chunk = next(c for c in range(min(64, b_per_w), 0, -8)
                 if b_per_w % c == 0)
    n_chunks = b_per_w // chunk
    mesh = plsc.VectorSubcoreMesh(core_axis_name="c", subcore_axis_name="s")

    @functools.partial(
        pl.kernel,
        mesh=mesh,
        out_type=(),
        scratch_types=[
            pltpu.VMEM((b_per_w,), jnp.int32),
            pltpu.VMEM((chunk, d), jnp.float32),
            pltpu.VMEM((chunk, d), jnp.float32),
            pltpu.SemaphoreType.DMA,
            pltpu.SemaphoreType.DMA,
            pltpu.SemaphoreType.DMA,
            pltpu.SemaphoreType.DMA,
        ],
    )
    def gather_kernel(table_hbm, idx_hbm, vq_hbm,
                      idx_v, rows0, rows1, gsem0, gsem1, wsem0, wsem1):
        wid = lax.axis_index("s") * info.num_cores + lax.axis_index("c")
        base = wid * b_per_w
        pltpu.sync_copy(idx_hbm.at[pl.ds(base, b_per_w)], idx_v)

        bufs = (rows0, rows1)
        gsems = (gsem0, gsem1)
        wsems = (wsem0, wsem1)

        def gath(c):
            return pltpu.async_copy(
                table_hbm.at[idx_v.at[pl.ds(c * chunk, chunk)]],
                bufs[c % 2], gsems[c % 2])

        writes = [None, None]
        gathers = [None, None]
        gathers[0] = gath(0)
        for c in range(n_chunks):
            nxt = c + 1
            if nxt < n_chunks:
                if writes[nxt % 2] is not None:
                    writes[nxt % 2].wait()
                gathers[nxt % 2] = gath(nxt)
            gathers[c % 2].wait()
            writes[c % 2] = pltpu.async_copy(
                bufs[c % 2],
                vq_hbm.at[pl.ds(slice_base + base + c * chunk, chunk)],
                wsems[c % 2])
        writes[(n_chunks - 1) % 2].wait()
        if n_chunks >= 2:
            writes[(n_chunks - 2) % 2].wait()

    return gather_kernel


def kernel(x, embed_weight):
    n, dim = x.shape
    k, _ = embed_weight.shape
    x2 = jnp.sum(x ** 2, axis=1).reshape(n // BN, 1, BN)
    z2 = jnp.sum(embed_weight ** 2, axis=1)
    vq_ref = jax.empty_ref(jax.ShapeDtypeStruct((n, dim), jnp.float32))
    losssum = jnp.float32(0.0)
    row_base = 0
    for ns in SLICE_ROWS:
        idx_s, ls = _vq_distances_slice(x, embed_weight, x2, z2, row_base, ns)
        losssum = losssum + ls[0, 0]
        _make_sc_gather_slice(dim, ns, row_base)(embed_weight, idx_s, vq_ref)
        row_base += ns
    loss = losssum * (1.25 / (n * dim))
    return (jax.freeze(vq_ref), loss)
